# baseline (device time: 121505 ns/iter reference)
import jax
import jax.numpy as jnp
from jax import lax
from jax.experimental import pallas as pl
from jax.experimental.pallas import tpu as pltpu

N_DEV = 32


def kernel(x, router_W, route_idx, expert_W):
    T, D = x.shape
    _, N_EXP = router_W.shape
    E_per, _, H = expert_W.shape
    K = E_per * D

    def body(x_ref, rw_ref, idx_ref, ew_ref, out_ref,
             comm_ref, send_sems, recv_sems):
        my = lax.axis_index("i")
        left = lax.rem(my + N_DEV - 1, N_DEV)
        right = lax.rem(my + 1, N_DEV)

        barrier_sem = pltpu.get_barrier_semaphore()
        for nbr in (left, right):
            pl.semaphore_signal(
                barrier_sem, inc=1,
                device_id=(nbr,), device_id_type=pl.DeviceIdType.MESH,
            )
        pl.semaphore_wait(barrier_sem, 2)

        xf = x_ref[...]
        scores = jnp.dot(xf, rw_ref[...],
                         preferred_element_type=jnp.float32)
        probs = jax.nn.softmax(scores, axis=-1)
        e_ids = lax.broadcasted_iota(jnp.int32, (T, N_EXP), 1)
        sel = (e_ids == idx_ref[:, 0:1]) | (e_ids == idx_ref[:, 1:2])
        w = jnp.where(sel, probs, 0.0)
        w = w / jnp.sum(w, axis=-1, keepdims=True)

        comm_ref[pl.ds(my, 1)] = (
            ew_ref[...].astype(jnp.bfloat16).reshape(1, K, H)
        )

        def contrib(o):
            row = lax.broadcasted_iota(jnp.int32, (N_EXP, E_per), 0)
            col = lax.broadcasted_iota(jnp.int32, (N_EXP, E_per), 1)
            onehot = (row == o * E_per + col).astype(jnp.float32)
            g = jnp.dot(w, onehot,
                        preferred_element_type=jnp.float32)
            xg = jnp.concatenate(
                [xf * g[:, 0:1], xf * g[:, 1:2]], axis=1
            ).astype(jnp.bfloat16)
            chunk = comm_ref[pl.ds(o, 1)].reshape(K, H)
            return jnp.dot(xg, chunk, preferred_element_type=jnp.float32)

        out_ref[...] = contrib(my)

        def hop(h, carry):
            s_send = lax.rem(my - h + 2 * N_DEV, N_DEV)
            s_recv = lax.rem(my - h - 1 + 2 * N_DEV, N_DEV)
            send = pltpu.make_async_remote_copy(
                src_ref=comm_ref.at[s_send],
                dst_ref=comm_ref.at[s_send],
                send_sem=send_sems.at[s_send],
                recv_sem=recv_sems.at[s_send],
                device_id=(right,),
                device_id_type=pl.DeviceIdType.MESH,
            )
            send.start()
            send.wait_send()
            recv = pltpu.make_async_remote_copy(
                src_ref=comm_ref.at[s_recv],
                dst_ref=comm_ref.at[s_recv],
                send_sem=send_sems.at[s_recv],
                recv_sem=recv_sems.at[s_recv],
                device_id=(left,),
                device_id_type=pl.DeviceIdType.MESH,
            )
            recv.wait_recv()
            out_ref[...] += contrib(s_recv)
            return carry

        lax.fori_loop(0, N_DEV - 1, hop, 0)

    return pl.pallas_call(
        body,
        out_shape=jax.ShapeDtypeStruct((T, H), jnp.float32),
        in_specs=[pl.BlockSpec(memory_space=pltpu.VMEM)] * 4,
        out_specs=pl.BlockSpec(memory_space=pltpu.VMEM),
        scratch_shapes=[
            pltpu.VMEM((N_DEV, K, H), jnp.bfloat16),
            pltpu.SemaphoreType.DMA((N_DEV,)),
            pltpu.SemaphoreType.DMA((N_DEV,)),
        ],
        compiler_params=pltpu.CompilerParams(collective_id=0),
    )(x, router_W, route_idx, expert_W)


# device time: 75995 ns/iter; 1.5989x vs baseline; 1.5989x over previous
import jax
import jax.numpy as jnp
from jax import lax
from jax.experimental import pallas as pl
from jax.experimental.pallas import tpu as pltpu

N_DEV = 32


def kernel(x, router_W, route_idx, expert_W):
    T, D = x.shape
    _, N_EXP = router_W.shape
    E_per, _, H = expert_W.shape
    K = E_per * D

    def body(x_ref, rw_ref, idx_ref, ew_ref, out_ref,
             comm_ref, send_sems, recv_sems):
        my = lax.axis_index("i")
        left = lax.rem(my + N_DEV - 1, N_DEV)
        right = lax.rem(my + 1, N_DEV)

        barrier_sem = pltpu.get_barrier_semaphore()

        def bar_signal(d, carry):
            pl.semaphore_signal(
                barrier_sem, inc=1,
                device_id=(lax.rem(my + d, N_DEV),),
                device_id_type=pl.DeviceIdType.MESH,
            )
            return carry

        lax.fori_loop(1, N_DEV, bar_signal, 0)
        pl.semaphore_wait(barrier_sem, N_DEV - 1)

        xf = x_ref[...]
        scores = jnp.dot(xf, rw_ref[...],
                         preferred_element_type=jnp.float32)
        probs = jax.nn.softmax(scores, axis=-1)
        e_ids = lax.broadcasted_iota(jnp.int32, (T, N_EXP), 1)
        sel = (e_ids == idx_ref[:, 0:1]) | (e_ids == idx_ref[:, 1:2])
        w = jnp.where(sel, probs, 0.0)
        w = w / jnp.sum(w, axis=-1, keepdims=True)

        comm_ref[pl.ds(my, 1)] = (
            ew_ref[...].astype(jnp.bfloat16).reshape(1, K, H)
        )

        def contrib(o):
            row = lax.broadcasted_iota(jnp.int32, (N_EXP, E_per), 0)
            col = lax.broadcasted_iota(jnp.int32, (N_EXP, E_per), 1)
            onehot = (row == o * E_per + col).astype(jnp.float32)
            g = jnp.dot(w, onehot,
                        preferred_element_type=jnp.float32)
            xg = jnp.concatenate(
                [xf * g[:, 0:1], xf * g[:, 1:2]], axis=1
            ).astype(jnp.bfloat16)
            chunk = comm_ref[pl.ds(o, 1)].reshape(K, H)
            return jnp.dot(xg, chunk, preferred_element_type=jnp.float32)

        def send_to(d, carry):
            dst = lax.rem(my + d, N_DEV)
            send = pltpu.make_async_remote_copy(
                src_ref=comm_ref.at[my],
                dst_ref=comm_ref.at[my],
                send_sem=send_sems.at[dst],
                recv_sem=recv_sems.at[my],
                device_id=(dst,),
                device_id_type=pl.DeviceIdType.MESH,
            )
            send.start()
            return carry

        lax.fori_loop(1, N_DEV, send_to, 0)

        out_ref[...] = contrib(my)

        def recv_from(d, carry):
            o = lax.rem(my + d, N_DEV)
            recv = pltpu.make_async_remote_copy(
                src_ref=comm_ref.at[o],
                dst_ref=comm_ref.at[o],
                send_sem=send_sems.at[o],
                recv_sem=recv_sems.at[o],
                device_id=(o,),
                device_id_type=pl.DeviceIdType.MESH,
            )
            recv.wait_recv()
            out_ref[...] += contrib(o)
            return carry

        lax.fori_loop(1, N_DEV, recv_from, 0)

        def drain(d, carry):
            dst = lax.rem(my + d, N_DEV)
            send = pltpu.make_async_remote_copy(
                src_ref=comm_ref.at[my],
                dst_ref=comm_ref.at[my],
                send_sem=send_sems.at[dst],
                recv_sem=recv_sems.at[my],
                device_id=(dst,),
                device_id_type=pl.DeviceIdType.MESH,
            )
            send.wait_send()
            return carry

        lax.fori_loop(1, N_DEV, drain, 0)

    return pl.pallas_call(
        body,
        out_shape=jax.ShapeDtypeStruct((T, H), jnp.float32),
        in_specs=[pl.BlockSpec(memory_space=pltpu.VMEM)] * 4,
        out_specs=pl.BlockSpec(memory_space=pltpu.VMEM),
        scratch_shapes=[
            pltpu.VMEM((N_DEV, K, H), jnp.bfloat16),
            pltpu.SemaphoreType.DMA((N_DEV,)),
            pltpu.SemaphoreType.DMA((N_DEV,)),
        ],
        compiler_params=pltpu.CompilerParams(collective_id=0),
    )(x, router_W, route_idx, expert_W)


# device time: 64426 ns/iter; 1.8860x vs baseline; 1.1796x over previous
import jax
import jax.numpy as jnp
from jax import lax
from jax.experimental import pallas as pl
from jax.experimental.pallas import tpu as pltpu

N_DEV = 32


def kernel(x, router_W, route_idx, expert_W):
    T, D = x.shape
    _, N_EXP = router_W.shape
    E_per, _, H = expert_W.shape
    K = E_per * D

    def body(x_ref, rw_ref, idx_ref, ew_ref, out_ref,
             comm_ref, xg_ref, send_sems, recv_sems):
        my = lax.axis_index("i")
        left = lax.rem(my + N_DEV - 1, N_DEV)
        right = lax.rem(my + 1, N_DEV)

        barrier_sem = pltpu.get_barrier_semaphore()

        def bar_signal(d, carry):
            pl.semaphore_signal(
                barrier_sem, inc=1,
                device_id=(lax.rem(my + d, N_DEV),),
                device_id_type=pl.DeviceIdType.MESH,
            )
            return carry

        lax.fori_loop(1, N_DEV, bar_signal, 0)
        pl.semaphore_wait(barrier_sem, N_DEV - 1)

        xf = x_ref[...]
        scores = jnp.dot(xf, rw_ref[...],
                         preferred_element_type=jnp.float32)
        probs = jax.nn.softmax(scores, axis=-1)
        e_ids = lax.broadcasted_iota(jnp.int32, (T, N_EXP), 1)
        sel = (e_ids == idx_ref[:, 0:1]) | (e_ids == idx_ref[:, 1:2])
        w = jnp.where(sel, probs, 0.0)
        w = w / jnp.sum(w, axis=-1, keepdims=True)

        comm_ref[pl.ds(my * K, K)] = (
            ew_ref[...].astype(jnp.bfloat16).reshape(K, H)
        )

        def send_to(d, carry):
            dst = lax.rem(my + d, N_DEV)
            send = pltpu.make_async_remote_copy(
                src_ref=comm_ref.at[pl.ds(my * K, K)],
                dst_ref=comm_ref.at[pl.ds(my * K, K)],
                send_sem=send_sems.at[dst],
                recv_sem=recv_sems.at[my],
                device_id=(dst,),
                device_id_type=pl.DeviceIdType.MESH,
            )
            send.start()
            return carry

        lax.fori_loop(1, N_DEV, send_to, 0)

        for e in range(N_EXP):
            xg_ref[:, e * D:(e + 1) * D] = (
                xf * w[:, e:e + 1]
            ).astype(jnp.bfloat16)

        def recv_from(d, carry):
            o = lax.rem(my + d, N_DEV)
            recv = pltpu.make_async_remote_copy(
                src_ref=comm_ref.at[pl.ds(o * K, K)],
                dst_ref=comm_ref.at[pl.ds(o * K, K)],
                send_sem=send_sems.at[o],
                recv_sem=recv_sems.at[o],
                device_id=(o,),
                device_id_type=pl.DeviceIdType.MESH,
            )
            recv.wait_recv()
            return carry

        lax.fori_loop(1, N_DEV, recv_from, 0)

        out_ref[...] = jnp.dot(xg_ref[...], comm_ref[...],
                               preferred_element_type=jnp.float32)

        def drain(d, carry):
            dst = lax.rem(my + d, N_DEV)
            send = pltpu.make_async_remote_copy(
                src_ref=comm_ref.at[pl.ds(my * K, K)],
                dst_ref=comm_ref.at[pl.ds(my * K, K)],
                send_sem=send_sems.at[dst],
                recv_sem=recv_sems.at[my],
                device_id=(dst,),
                device_id_type=pl.DeviceIdType.MESH,
            )
            send.wait_send()
            return carry

        lax.fori_loop(1, N_DEV, drain, 0)

    return pl.pallas_call(
        body,
        out_shape=jax.ShapeDtypeStruct((T, H), jnp.float32),
        in_specs=[pl.BlockSpec(memory_space=pltpu.VMEM)] * 4,
        out_specs=pl.BlockSpec(memory_space=pltpu.VMEM),
        scratch_shapes=[
            pltpu.VMEM((N_DEV * K, H), jnp.bfloat16),
            pltpu.VMEM((T, N_EXP * D), jnp.bfloat16),
            pltpu.SemaphoreType.DMA((N_DEV,)),
            pltpu.SemaphoreType.DMA((N_DEV,)),
        ],
        compiler_params=pltpu.CompilerParams(collective_id=0),
    )(x, router_W, route_idx, expert_W)
